# baseline (speedup 1.0000x reference)
import functools

import jax
import jax.numpy as jnp
from jax import lax
from jax.experimental import pallas as pl
from jax.experimental.pallas import tpu as pltpu
from jax.experimental.pallas import tpu_sc as plsc

_N = 10000
_E = 320000
_D = 128
_A = 5
_AP = 8
_NEG = -1e30

_NC = 2
_NS = 16
_NW = _NC * _NS
_CH = 80
_KS = 5
_BB = _CH * _KS
_NB = 5
_SCK = _BB * _NB
_NZ = 10240
_TPN = _NZ // _NS

_E1 = 192000
_E2 = _E - _E1

_EB = 2560


def _k1_body(x_ref, attn_w_ref, attn_b_ref, anchor_ref, ew1_ref, ew2_ref,
             eb_ref, npx_ref, s1_ref, s2_ref):
    x = x_ref[...]
    sc = jnp.dot(x, attn_w_ref[...], preferred_element_type=jnp.float32)
    sc = sc + attn_b_ref[...]
    m = jnp.max(sc, axis=1, keepdims=True)
    e = jnp.exp(sc - m)
    w = e / jnp.sum(e, axis=1, keepdims=True)
    npx_ref[...] = x + jnp.dot(w, anchor_ref[...],
                               preferred_element_type=jnp.float32)
    dn = (((0,), (1,)), ((), ()))
    s1_ref[...] = lax.dot_general(ew1_ref[...], x, dn,
                                  preferred_element_type=jnp.float32) + eb_ref[...]
    s2_ref[...] = lax.dot_general(ew2_ref[...], x, dn,
                                  preferred_element_type=jnp.float32)


def _run_k1(x, attn_wp, attn_bp, anchor_np, ew1, ew2, ebp):
    return pl.pallas_call(
        _k1_body,
        out_shape=[
            jax.ShapeDtypeStruct((_N, _D), jnp.float32),
            jax.ShapeDtypeStruct((_AP, _N), jnp.float32),
            jax.ShapeDtypeStruct((_AP, _N), jnp.float32),
        ],
    )(x, attn_wp, attn_bp, anchor_np, ew1, ew2, ebp)


def _make_sc_body(nsc):
  ew = nsc * _SCK

  def _sc_body(s1_hbm, s2_hbm, src_hbm, dst_hbm, bt_hbm, agg_hbm,
               table, idx_s, idx_d, bbuf, bbt, aggsh, sem):
    cid = lax.axis_index("c")
    sid = lax.axis_index("s")
    wid = sid * _NC + cid
    iota = lax.iota(jnp.int32, 16)
    zv = jnp.zeros((16,), jnp.float32)
    ones = jnp.ones((16,), jnp.float32)

    for g in range(_BB // 16):
        rows = g * 16 + iota
        for a in range(_AP):
            plsc.store_scatter(bbuf, [rows, jnp.full((16,), a, jnp.int32)], zv)

    def _zero(r, c):
        pltpu.sync_copy(bbuf.at[pl.ds(0, 320)],
                        aggsh.at[pl.ds(sid * _TPN + r * 320, 320)])
        return c
    lax.fori_loop(0, _TPN // 320, _zero, 0)

    for g in range(_BB // 16):
        rows = g * 16 + iota
        plsc.store_scatter(bbuf, [rows, jnp.full((16,), _A, jnp.int32)], ones)

    def _zpad(g, c):
        for a in range(_A, _AP):
            bbt[a, pl.ds(g * 16, 16)] = zv
        return c
    lax.fori_loop(0, _SCK // 16, _zpad, 0)

    pltpu.sync_copy(s1_hbm.at[pl.ds(0, _A)], table.at[pl.ds(0, _A)])
    pltpu.sync_copy(s2_hbm.at[pl.ds(0, _A)], table.at[pl.ds(_A, _A)])
    plsc.subcore_barrier()

    def _chunk(j, c):
        def _blk(b2, c2):
            pltpu.sync_copy(
                src_hbm.at[wid, pl.ds(j * _KS * _NB + b2 * _KS, _KS)], idx_s)
            pltpu.sync_copy(
                dst_hbm.at[wid, pl.ds(j * _KS * _NB + b2 * _KS, _KS)], idx_d)
            for k in range(_KS):
                for g in range(_CH // 16):
                    col = b2 * _BB + k * _CH + g * 16
                    rb = k * _CH + g * 16 + iota
                    iv_s = idx_s[k, pl.ds(g * 16, 16)]
                    iv_d = idx_d[k, pl.ds(g * 16, 16)]
                    logit = []
                    for a in range(_A):
                        v = (plsc.load_gather(
                                table, [jnp.full((16,), a, jnp.int32), iv_s])
                             + plsc.load_gather(
                                table, [jnp.full((16,), _A + a, jnp.int32), iv_d]))
                        logit.append(jnp.where(v > 0, v, v * 0.01))
                    m = logit[0]
                    for a in range(1, _A):
                        m = jnp.maximum(m, logit[a])
                    ex = [jnp.exp(v - m) for v in logit]
                    s = ex[0]
                    for a in range(1, _A):
                        s = s + ex[a]
                    inv = 1.0 / s
                    for a in range(_A):
                        ba = ex[a] * inv
                        bbt[a, pl.ds(col, 16)] = ba
                        plsc.store_scatter(
                            bbuf, [rb, jnp.full((16,), a, jnp.int32)], ba)
            for k in range(_KS):
                pltpu.sync_copy(bbuf.at[pl.ds(k * _CH, _CH)],
                                aggsh.at[idx_s.at[k]], add=True)
            return c2
        lax.fori_loop(0, _NB, _blk, 0)
        pltpu.sync_copy(bbt, bt_hbm.at[:, pl.ds(wid * ew + j * _SCK, _SCK)])
        return c
    lax.fori_loop(0, nsc, _chunk, 0)

    plsc.subcore_barrier()
    pltpu.sync_copy(aggsh.at[pl.ds(sid * _TPN, _TPN)],
                    agg_hbm.at[cid, pl.ds(sid * _TPN, _TPN)])

  return _sc_body


def _run_k2(s1, s2, src3, dst3, ecount):
    nsc = ecount // _NW // _SCK
    mesh = plsc.VectorSubcoreMesh(core_axis_name="c", subcore_axis_name="s",
                                  num_cores=_NC, num_subcores=_NS)
    fn = pl.kernel(
        _make_sc_body(nsc),
        out_type=[
            jax.ShapeDtypeStruct((_AP, ecount), jnp.float32),
            jax.ShapeDtypeStruct((_NC, _NZ, _AP), jnp.float32),
        ],
        mesh=mesh,
        scratch_types=[
            pltpu.VMEM((2 * _A, _N), jnp.float32),
            pltpu.VMEM((_KS, _CH), jnp.int32),
            pltpu.VMEM((_KS, _CH), jnp.int32),
            pltpu.VMEM((_BB, _AP), jnp.float32),
            pltpu.VMEM((_AP, _SCK), jnp.float32),
            pltpu.VMEM_SHARED((_NZ, _AP), jnp.float32),
            pltpu.SemaphoreType.DMA,
        ],
        compiler_params=pltpu.CompilerParams(needs_layout_passes=False,
                                             use_tc_tiling_on_sc=False),
    )
    return fn(s1, s2, src3, dst3)


def _k3_body(bt_ref, anchor_ref, out_ref):
    out_ref[...] = lax.dot_general(bt_ref[...], anchor_ref[...],
                                   (((0,), (0,)), ((), ())),
                                   preferred_element_type=jnp.float32)


def _run_k3(bt, anchor_ep):
    return pl.pallas_call(
        _k3_body,
        grid=(_E // _EB,),
        in_specs=[
            pl.BlockSpec((_AP, _EB), lambda i: (0, i)),
            pl.BlockSpec((_AP, _D), lambda i: (0, 0)),
        ],
        out_specs=pl.BlockSpec((_EB, _D), lambda i: (i, 0)),
        out_shape=jax.ShapeDtypeStruct((_E, _D), jnp.float32),
    )(bt, anchor_ep)


def _k4_body(agg_ref, npx_ref, r1_ref, r2_ref, w1_ref, b1_ref,
             w2_ref, b2_ref, anchor_ref, out_ref):
    agg = agg_ref[...]
    aggf = (agg[0] + agg[1])[:_N]
    deg = aggf[:, _A:_A + 1]
    dmax = jnp.max(deg)
    dsum = jnp.sum(deg)
    dmean = dsum / float(_N)
    t0 = deg / (dmax + 1e-6)
    t1 = r1_ref[...] * 0.5 + 0.25
    t2 = deg / (dsum + 1e-6)
    t3 = jax.nn.sigmoid(deg - dmean)
    t4 = r2_ref[...]
    tf = jnp.concatenate(
        [t0, t1, t2, t3, t4, jnp.zeros((_N, _AP - _A), jnp.float32)], axis=1)
    h = jnp.maximum(
        jnp.dot(tf, w1_ref[...], preferred_element_type=jnp.float32)
        + b1_ref[...], 0.0)
    te = jnp.dot(h, w2_ref[...], preferred_element_type=jnp.float32)
    te = te + b2_ref[...]
    eagg = jnp.dot(aggf, anchor_ref[...], preferred_element_type=jnp.float32)
    out_ref[...] = npx_ref[...] + te * eagg


def _run_k4(agg, npx, r1, r2, w1p, b1, w2, b2, anchor_ep):
    return pl.pallas_call(
        _k4_body,
        out_shape=jax.ShapeDtypeStruct((_N, _D), jnp.float32),
    )(agg, npx, r1, r2, w1p, b1, w2, b2, anchor_ep)


def kernel(x, edge_index, layer, node_anchor, attn_W, attn_b, edge_anchor,
           edge_W, edge_b, topo_W1, topo_b1, topo_W2, topo_b2):
    f32 = jnp.float32
    x = x.astype(f32)
    attn_wp = jnp.zeros((_D, _AP), f32).at[:, :_A].set(attn_W)
    attn_bp = jnp.full((1, _AP), _NEG, f32).at[0, :_A].set(attn_b)
    anchor_np = jnp.zeros((_AP, _D), f32).at[:_A].set(node_anchor)
    ew1 = jnp.zeros((_D, _AP), f32).at[:, :_A].set(edge_W[:_D])
    ew2 = jnp.zeros((_D, _AP), f32).at[:, :_A].set(edge_W[_D:])
    ebp = jnp.zeros((_AP, 1), f32).at[:_A, 0].set(edge_b)
    anchor_ep = jnp.zeros((_AP, _D), f32).at[:_A].set(edge_anchor)
    hdim = topo_W1.shape[1]
    w1p = jnp.zeros((_AP, hdim), f32).at[:_A].set(topo_W1)
    b1 = topo_b1.reshape(1, hdim).astype(f32)
    b2 = topo_b2.reshape(1, _D).astype(f32)
    src3 = edge_index[0].reshape(_NW, _E // _NW // _CH, _CH)
    dst3 = edge_index[1].reshape(_NW, _E // _NW // _CH, _CH)
    kr = jax.random.key(42)
    r1 = jax.random.uniform(jax.random.fold_in(kr, 1), (_N,), f32)
    r2 = jax.random.uniform(jax.random.fold_in(kr, 2), (_N,), f32)
    r1 = r1.reshape(_N, 1)
    r2 = r2.reshape(_N, 1)

    npx, s1, s2 = _run_k1(x, attn_wp, attn_bp, anchor_np, ew1, ew2, ebp)
    bt, agg = _run_k2(s1, s2, src3, dst3, _E)
    edge_prompt = _run_k3(bt, anchor_ep)
    final_x = _run_k4(agg, npx, r1, r2, w1p, b1,
                      topo_W2.astype(f32), b2, anchor_ep)
    return (final_x, edge_prompt)


# speedup vs baseline: 1.0323x; 1.0323x over previous
import functools

import jax
import jax.numpy as jnp
from jax import lax
from jax.experimental import pallas as pl
from jax.experimental.pallas import tpu as pltpu
from jax.experimental.pallas import tpu_sc as plsc

_N = 10000
_E = 320000
_D = 128
_A = 5
_AP = 8
_NEG = -1e30

_NC = 2
_NS = 16
_NW = _NC * _NS
_CH = 80
_KS = 5
_BB = _CH * _KS
_NB = 5
_SCK = _BB * _NB
_NZ = 10240
_TPN = _NZ // _NS

_E1 = 192000
_E2 = _E - _E1

_EB = 2560


def _k1_body(x_ref, attn_w_ref, attn_b_ref, anchor_ref, ew1_ref, ew2_ref,
             eb_ref, npx_ref, s1_ref, s2_ref):
    x = x_ref[...]
    sc = jnp.dot(x, attn_w_ref[...], preferred_element_type=jnp.float32)
    sc = sc + attn_b_ref[...]
    m = jnp.max(sc, axis=1, keepdims=True)
    e = jnp.exp(sc - m)
    w = e / jnp.sum(e, axis=1, keepdims=True)
    npx_ref[...] = x + jnp.dot(w, anchor_ref[...],
                               preferred_element_type=jnp.float32)
    dn = (((0,), (1,)), ((), ()))
    s1_ref[...] = lax.dot_general(ew1_ref[...], x, dn,
                                  preferred_element_type=jnp.float32) + eb_ref[...]
    s2_ref[...] = lax.dot_general(ew2_ref[...], x, dn,
                                  preferred_element_type=jnp.float32)


def _run_k1(x, attn_wp, attn_bp, anchor_np, ew1, ew2, ebp):
    return pl.pallas_call(
        _k1_body,
        out_shape=[
            jax.ShapeDtypeStruct((_N, _D), jnp.float32),
            jax.ShapeDtypeStruct((_AP, _N), jnp.float32),
            jax.ShapeDtypeStruct((_AP, _N), jnp.float32),
        ],
    )(x, attn_wp, attn_bp, anchor_np, ew1, ew2, ebp)


def _make_sc_body(nsc):
  ew = nsc * _SCK

  def _sc_body(s1_hbm, s2_hbm, src_hbm, dst_hbm, bt_hbm, agg_hbm,
               table, idx_s, idx_d, bbuf, bbt, aggsh, sem):
    cid = lax.axis_index("c")
    sid = lax.axis_index("s")
    wid = sid * _NC + cid
    iota = lax.iota(jnp.int32, 16)
    zv = jnp.zeros((16,), jnp.float32)
    ones = jnp.ones((16,), jnp.float32)

    for g in range(_BB // 16):
        rows = g * 16 + iota
        for a in range(_AP):
            plsc.store_scatter(bbuf, [rows, jnp.full((16,), a, jnp.int32)], zv)

    def _zero(r, c):
        pltpu.sync_copy(bbuf.at[pl.ds(0, 320)],
                        aggsh.at[pl.ds(sid * _TPN + r * 320, 320)])
        return c
    lax.fori_loop(0, _TPN // 320, _zero, 0)

    for g in range(_BB // 16):
        rows = g * 16 + iota
        plsc.store_scatter(bbuf, [rows, jnp.full((16,), _A, jnp.int32)], ones)

    def _zpad(g, c):
        for a in range(_A, _AP):
            bbt[a, pl.ds(g * 16, 16)] = zv
        return c
    lax.fori_loop(0, _BB // 16, _zpad, 0)

    pltpu.sync_copy(s1_hbm.at[pl.ds(0, _A)], table.at[pl.ds(0, _A)])
    pltpu.sync_copy(s2_hbm.at[pl.ds(0, _A)], table.at[pl.ds(_A, _A)])
    plsc.subcore_barrier()

    def _chunk(j, c):
            pltpu.sync_copy(src_hbm.at[wid, pl.ds(j * _KS, _KS)], idx_s)
            pltpu.sync_copy(dst_hbm.at[wid, pl.ds(j * _KS, _KS)], idx_d)
            for k in range(_KS):
                for g in range(_CH // 16):
                    col = k * _CH + g * 16
                    rb = col + iota
                    iv_s = idx_s[k, pl.ds(g * 16, 16)]
                    iv_d = idx_d[k, pl.ds(g * 16, 16)]
                    logit = []
                    for a in range(_A):
                        v = (plsc.load_gather(
                                table, [jnp.full((16,), a, jnp.int32), iv_s])
                             + plsc.load_gather(
                                table, [jnp.full((16,), _A + a, jnp.int32), iv_d]))
                        logit.append(jnp.where(v > 0, v, v * 0.01))
                    m = logit[0]
                    for a in range(1, _A):
                        m = jnp.maximum(m, logit[a])
                    ex = [jnp.exp(v - m) for v in logit]
                    s = ex[0]
                    for a in range(1, _A):
                        s = s + ex[a]
                    inv = 1.0 / s
                    for a in range(_A):
                        ba = ex[a] * inv
                        bbt[a, pl.ds(col, 16)] = ba
                        plsc.store_scatter(
                            bbuf, [rb, jnp.full((16,), a, jnp.int32)], ba)
            pltpu.sync_copy(bbt, bt_hbm.at[:, pl.ds(wid * ew + j * _BB, _BB)])
            for k in range(_KS):
                pltpu.sync_copy(bbuf.at[pl.ds(k * _CH, _CH)],
                                aggsh.at[idx_s.at[k]], add=True)
            return c
    lax.fori_loop(0, ew // _BB, _chunk, 0)

    plsc.subcore_barrier()
    pltpu.sync_copy(aggsh.at[pl.ds(sid * _TPN, _TPN)],
                    agg_hbm.at[cid, pl.ds(sid * _TPN, _TPN)])

  return _sc_body


def _run_k2(s1, s2, src3, dst3, ecount):
    nsc = ecount // _NW // _SCK
    mesh = plsc.VectorSubcoreMesh(core_axis_name="c", subcore_axis_name="s",
                                  num_cores=_NC, num_subcores=_NS)
    fn = pl.kernel(
        _make_sc_body(nsc),
        out_type=[
            jax.ShapeDtypeStruct((_AP, ecount), jnp.float32),
            jax.ShapeDtypeStruct((_NC, _NZ, _AP), jnp.float32),
        ],
        mesh=mesh,
        scratch_types=[
            pltpu.VMEM((2 * _A, _N), jnp.float32),
            pltpu.VMEM((_KS, _CH), jnp.int32),
            pltpu.VMEM((_KS, _CH), jnp.int32),
            pltpu.VMEM((_BB, _AP), jnp.float32),
            pltpu.VMEM((_AP, _BB), jnp.float32),
            pltpu.VMEM_SHARED((_NZ, _AP), jnp.float32),
            pltpu.SemaphoreType.DMA,
        ],
        compiler_params=pltpu.CompilerParams(needs_layout_passes=False,
                                             use_tc_tiling_on_sc=False),
    )
    return fn(s1, s2, src3, dst3)


def _k3_body(bt_ref, anchor_ref, out_ref):
    out_ref[...] = lax.dot_general(bt_ref[...], anchor_ref[...],
                                   (((0,), (0,)), ((), ())),
                                   preferred_element_type=jnp.float32)


def _run_k3(bt, anchor_ep):
    return pl.pallas_call(
        _k3_body,
        grid=(_E // _EB,),
        in_specs=[
            pl.BlockSpec((_AP, _EB), lambda i: (0, i)),
            pl.BlockSpec((_AP, _D), lambda i: (0, 0)),
        ],
        out_specs=pl.BlockSpec((_EB, _D), lambda i: (i, 0)),
        out_shape=jax.ShapeDtypeStruct((_E, _D), jnp.float32),
    )(bt, anchor_ep)


def _k4_body(agg_ref, npx_ref, r1_ref, r2_ref, w1_ref, b1_ref,
             w2_ref, b2_ref, anchor_ref, out_ref):
    agg = agg_ref[...]
    aggf = (agg[0] + agg[1])[:_N]
    deg = aggf[:, _A:_A + 1]
    dmax = jnp.max(deg)
    dsum = jnp.sum(deg)
    dmean = dsum / float(_N)
    t0 = deg / (dmax + 1e-6)
    t1 = r1_ref[...] * 0.5 + 0.25
    t2 = deg / (dsum + 1e-6)
    t3 = jax.nn.sigmoid(deg - dmean)
    t4 = r2_ref[...]
    tf = jnp.concatenate(
        [t0, t1, t2, t3, t4, jnp.zeros((_N, _AP - _A), jnp.float32)], axis=1)
    h = jnp.maximum(
        jnp.dot(tf, w1_ref[...], preferred_element_type=jnp.float32)
        + b1_ref[...], 0.0)
    te = jnp.dot(h, w2_ref[...], preferred_element_type=jnp.float32)
    te = te + b2_ref[...]
    eagg = jnp.dot(aggf, anchor_ref[...], preferred_element_type=jnp.float32)
    out_ref[...] = npx_ref[...] + te * eagg


def _run_k4(agg, npx, r1, r2, w1p, b1, w2, b2, anchor_ep):
    return pl.pallas_call(
        _k4_body,
        out_shape=jax.ShapeDtypeStruct((_N, _D), jnp.float32),
    )(agg, npx, r1, r2, w1p, b1, w2, b2, anchor_ep)


def kernel(x, edge_index, layer, node_anchor, attn_W, attn_b, edge_anchor,
           edge_W, edge_b, topo_W1, topo_b1, topo_W2, topo_b2):
    f32 = jnp.float32
    x = x.astype(f32)
    attn_wp = jnp.zeros((_D, _AP), f32).at[:, :_A].set(attn_W)
    attn_bp = jnp.full((1, _AP), _NEG, f32).at[0, :_A].set(attn_b)
    anchor_np = jnp.zeros((_AP, _D), f32).at[:_A].set(node_anchor)
    ew1 = jnp.zeros((_D, _AP), f32).at[:, :_A].set(edge_W[:_D])
    ew2 = jnp.zeros((_D, _AP), f32).at[:, :_A].set(edge_W[_D:])
    ebp = jnp.zeros((_AP, 1), f32).at[:_A, 0].set(edge_b)
    anchor_ep = jnp.zeros((_AP, _D), f32).at[:_A].set(edge_anchor)
    hdim = topo_W1.shape[1]
    w1p = jnp.zeros((_AP, hdim), f32).at[:_A].set(topo_W1)
    b1 = topo_b1.reshape(1, hdim).astype(f32)
    b2 = topo_b2.reshape(1, _D).astype(f32)
    src3 = edge_index[0].reshape(_NW, _E // _NW // _CH, _CH)
    dst3 = edge_index[1].reshape(_NW, _E // _NW // _CH, _CH)
    kr = jax.random.key(42)
    r1 = jax.random.uniform(jax.random.fold_in(kr, 1), (_N,), f32)
    r2 = jax.random.uniform(jax.random.fold_in(kr, 2), (_N,), f32)
    r1 = r1.reshape(_N, 1)
    r2 = r2.reshape(_N, 1)

    npx, s1, s2 = _run_k1(x, attn_wp, attn_bp, anchor_np, ew1, ew2, ebp)
    bt, agg = _run_k2(s1, s2, src3, dst3, _E)
    edge_prompt = _run_k3(bt, anchor_ep)
    final_x = _run_k4(agg, npx, r1, r2, w1p, b1,
                      topo_W2.astype(f32), b2, anchor_ep)
    return (final_x, edge_prompt)
